# baseline (speedup 1.0000x reference)
import functools

import jax
import jax.numpy as jnp
from jax import lax
from jax.experimental import pallas as pl
from jax.experimental.pallas import tpu as pltpu
from jax.experimental.pallas import tpu_sc as plsc

F = 26
V = 100000
D = 16
B = 16384

NC = 2
NS = 16
NW = NC * NS

BPW = B // NW
OPS = 4
SLW = OPS * F
NSL = BPW // OPS
NBUF = 4


def _encoder_body(idx_hbm, t0, t1, t2, t3, o0, o1, o2, o3,
                  idx_v, rows_v, out_v, sem):
    wid = lax.axis_index("s") * NC + lax.axis_index("c")
    base = wid * BPW

    pltpu.sync_copy(idx_hbm.at[wid], idx_v)

    for tab, out in ((t0, o0), (t1, o1), (t2, o2), (t3, o3)):
        for b in range(NBUF):
            pltpu.async_copy(tab.at[idx_v.at[b]], rows_v.at[b], sem)

        def body(g, carry, tab=tab):
            for b in range(NBUF):
                j = g * NBUF + b
                pltpu.make_async_copy(tab.at[idx_v.at[0]], rows_v.at[b],
                                      sem).wait()
                for o in range(OPS):
                    acc = rows_v[b, o * F, :]
                    for f in range(1, F):
                        acc = acc + rows_v[b, o * F + f, :]
                    out_v[j * OPS + o, :] = acc
                nj = j + NBUF

                @pl.when(nj < NSL)
                def _(nj=nj, b=b, tab=tab):
                    pltpu.async_copy(tab.at[idx_v.at[nj]], rows_v.at[b], sem)

            return carry

        lax.fori_loop(0, NSL // NBUF, body, None)
        pltpu.sync_copy(out_v, out.at[pl.ds(base, BPW)])


@functools.partial(jax.jit, static_argnums=())
def kernel(x, r_tab, i_tab, j_tab, k_tab):
    offs = (jnp.arange(F, dtype=jnp.int32) * V)[None, :]
    idx = (x.astype(jnp.int32) + offs).reshape(NW, NSL, SLW)
    tabs = [t.reshape(F * V, D) for t in (r_tab, i_tab, j_tab, k_tab)]

    out_sds = jax.ShapeDtypeStruct((B, D), jnp.float32)
    enc = pl.kernel(
        _encoder_body,
        out_type=(out_sds, out_sds, out_sds, out_sds),
        mesh=plsc.VectorSubcoreMesh(core_axis_name="c", subcore_axis_name="s",
                                    num_cores=NC, num_subcores=NS),
        compiler_params=pltpu.CompilerParams(use_tc_tiling_on_sc=False),
        scratch_types=[
            pltpu.VMEM((NSL, SLW), jnp.int32),
            pltpu.VMEM((NBUF, SLW, D), jnp.float32),
            pltpu.VMEM((BPW, D), jnp.float32),
            pltpu.SemaphoreType.DMA,
        ],
    )
    return enc(idx, *tabs)


# speedup vs baseline: 1.0170x; 1.0170x over previous
import functools

import jax
import jax.numpy as jnp
from jax import lax
from jax.experimental import pallas as pl
from jax.experimental.pallas import tpu as pltpu
from jax.experimental.pallas import tpu_sc as plsc

F = 26
V = 100000
D = 16
B = 16384

NC = 2
NS = 16
NW = NC * NS

BPW = B // NW
CO = 64
CR = CO * F
NCH = BPW // CO
NBUF = 2


def _encoder_body(idx_hbm, t0, t1, t2, t3, o0, o1, o2, o3,
                  idx_v, rows_v, out_v, sem):
    wid = lax.axis_index("s") * NC + lax.axis_index("c")
    base = wid * BPW

    pltpu.sync_copy(idx_hbm.at[wid], idx_v)

    for tab, out in ((t0, o0), (t1, o1), (t2, o2), (t3, o3)):
        for b in range(NBUF):
            pltpu.async_copy(tab.at[idx_v.at[pl.ds(b * CR, CR)]],
                             rows_v.at[b], sem)

        def outer(g, carry, tab=tab):
            for b in range(NBUF):
                j = g * NBUF + b
                pltpu.make_async_copy(tab.at[idx_v.at[pl.ds(0, CR)]],
                                      rows_v.at[b], sem).wait()

                def inner(o, c2, b=b, j=j):
                    rbase = o * F
                    acc = rows_v[b, rbase, :]
                    for f in range(1, F):
                        acc = acc + rows_v[b, rbase + f, :]
                    out_v[j * CO + o, :] = acc
                    return c2

                lax.fori_loop(0, CO, inner, None)
                nj = j + NBUF

                @pl.when(nj < NCH)
                def _(nj=nj, b=b, tab=tab):
                    pltpu.async_copy(tab.at[idx_v.at[pl.ds(nj * CR, CR)]],
                                     rows_v.at[b], sem)

            return carry

        lax.fori_loop(0, NCH // NBUF, outer, None)
        pltpu.sync_copy(out_v, out.at[pl.ds(base, BPW)])


@functools.partial(jax.jit, static_argnums=())
def kernel(x, r_tab, i_tab, j_tab, k_tab):
    offs = (jnp.arange(F, dtype=jnp.int32) * V)[None, :]
    idx = (x.astype(jnp.int32) + offs).reshape(NW, BPW * F)
    tabs = [t.reshape(F * V, D) for t in (r_tab, i_tab, j_tab, k_tab)]

    out_sds = jax.ShapeDtypeStruct((B, D), jnp.float32)
    enc = pl.kernel(
        _encoder_body,
        out_type=(out_sds, out_sds, out_sds, out_sds),
        mesh=plsc.VectorSubcoreMesh(core_axis_name="c", subcore_axis_name="s",
                                    num_cores=NC, num_subcores=NS),
        compiler_params=pltpu.CompilerParams(use_tc_tiling_on_sc=False),
        scratch_types=[
            pltpu.VMEM((BPW * F,), jnp.int32),
            pltpu.VMEM((NBUF, CR, D), jnp.float32),
            pltpu.VMEM((BPW, D), jnp.float32),
            pltpu.SemaphoreType.DMA,
        ],
    )
    return enc(idx, *tabs)


# speedup vs baseline: 1.0182x; 1.0012x over previous
import functools

import jax
import jax.numpy as jnp
from jax import lax
from jax.experimental import pallas as pl
from jax.experimental.pallas import tpu as pltpu
from jax.experimental.pallas import tpu_sc as plsc

F = 26
V = 100000
D = 16
B = 16384

NC = 2
NS = 16
NW = NC * NS

BPW = B // NW
CO = 64
G = BPW // CO
NBUF = 2


def _encoder_body(xt_hbm, t0, t1, t2, t3, o0, o1, o2, o3,
                  xv, rows_v, acc_v, semx, semg0, semg1):
    wid = lax.axis_index("s") * NC + lax.axis_index("c")
    base = wid * BPW
    semg = (semg0, semg1)

    for f in range(F):
        pltpu.async_copy(xt_hbm.at[f, pl.ds(base, BPW)], xv.at[f], semx)
    for f in range(F):
        pltpu.make_async_copy(xt_hbm.at[0, pl.ds(base, BPW)],
                              xv.at[0], semx).wait()

    def enqueue_group(tab, g, b):
        for f in range(F):
            pltpu.async_copy(
                tab.at[f].at[xv.at[f, pl.ds(g * CO, CO)]],
                rows_v.at[b, f], semg[b])

    def wait_group(tab, b):
        for f in range(F):
            pltpu.make_async_copy(tab.at[0].at[xv.at[0, pl.ds(0, CO)]],
                                  rows_v.at[b, 0], semg[b]).wait()

    for tab, out in ((t0, o0), (t1, o1), (t2, o2), (t3, o3)):
        for b in range(NBUF):
            enqueue_group(tab, b, b)

        def outer(gp, carry, tab=tab):
            for b in range(NBUF):
                g = gp * NBUF + b
                wait_group(tab, b)

                def inner(o, c2, b=b, g=g):
                    acc = rows_v[b, 0, o, :]
                    for f in range(1, F):
                        acc = acc + rows_v[b, f, o, :]
                    acc_v[g * CO + o, :] = acc
                    return c2

                lax.fori_loop(0, CO, inner, None)
                ng = g + NBUF

                @pl.when(ng < G)
                def _(ng=ng, b=b, tab=tab):
                    enqueue_group(tab, ng, b)

            return carry

        lax.fori_loop(0, G // NBUF, outer, None)
        pltpu.sync_copy(acc_v, out.at[pl.ds(base, BPW)])


@functools.partial(jax.jit, static_argnums=())
def kernel(x, r_tab, i_tab, j_tab, k_tab):
    xt = jnp.transpose(x.astype(jnp.int32))

    out_sds = jax.ShapeDtypeStruct((B, D), jnp.float32)
    enc = pl.kernel(
        _encoder_body,
        out_type=(out_sds, out_sds, out_sds, out_sds),
        mesh=plsc.VectorSubcoreMesh(core_axis_name="c", subcore_axis_name="s",
                                    num_cores=NC, num_subcores=NS),
        compiler_params=pltpu.CompilerParams(use_tc_tiling_on_sc=False),
        scratch_types=[
            pltpu.VMEM((F, BPW), jnp.int32),
            pltpu.VMEM((NBUF, F, CO, D), jnp.float32),
            pltpu.VMEM((BPW, D), jnp.float32),
            pltpu.SemaphoreType.DMA,
            pltpu.SemaphoreType.DMA,
            pltpu.SemaphoreType.DMA,
        ],
    )
    return enc(xt, r_tab, i_tab, j_tab, k_tab)
